# baseline (speedup 1.0000x reference)
import functools
import math

import jax
import jax.numpy as jnp
from jax import lax
from jax.experimental import pallas as pl
from jax.experimental.pallas import tpu as pltpu
from jax.experimental.pallas import tpu_sc as plsc

EMBED_DIM = 64
BATCH = 4096
HIST = 200
VOCAB = 1000000
NUM_CORES = 2
NUM_SUBCORES = 16
NW = NUM_CORES * NUM_SUBCORES
BW = BATCH // NW
SCALE = math.sqrt(EMBED_DIM)
L = 16

_mesh = plsc.VectorSubcoreMesh(core_axis_name="c", subcore_axis_name="s")


@functools.partial(
    pl.kernel,
    mesh=_mesh,
    compiler_params=pltpu.CompilerParams(
        use_tc_tiling_on_sc=False, needs_layout_passes=False),
    out_type=jax.ShapeDtypeStruct((HIST, 8, NW, 8, 128), jnp.float32),
    scratch_types=[
        pltpu.VMEM((2, BW), jnp.int32),
        pltpu.VMEM((2, BW, EMBED_DIM), jnp.float32),
        pltpu.VMEM((2, 8, 8, 128), jnp.float32),
        pltpu.SemaphoreType.DMA,
        pltpu.SemaphoreType.DMA,
        pltpu.SemaphoreType.DMA,
    ],
)
def _emb_lookup(idx_hbm, table_hbm, out_hbm, idx_v, rows_v, outb_v,
                isem, gsem, osem):
    wid = lax.axis_index("s") * NUM_CORES + lax.axis_index("c")
    bbase = wid * BW

    def fetch_idx(h, buf):
        pltpu.async_copy(
            idx_hbm.at[pl.ds(h * BATCH + bbase, BW)], idx_v.at[buf], isem
        ).wait()

    def start_gather(buf):
        pltpu.async_copy(table_hbm.at[idx_v.at[buf]], rows_v.at[buf], gsem)

    def wait_gather(buf):
        pltpu.make_async_copy(
            table_hbm.at[idx_v.at[buf]], rows_v.at[buf], gsem
        ).wait()

    def slab_compute(h, buf):
        rows = rows_v.at[buf]
        for g in range(BW // L):
            sl = pl.ds(g * L, L)
            rowi = lax.iota(jnp.int32, L) + g * L
            for d in range(EMBED_DIM):
                vals = plsc.load_gather(
                    rows, [rowi, jnp.full((L,), d, jnp.int32)])
                outb_v[buf, d // 8, d % 8, sl] = vals * SCALE
        pltpu.async_copy(
            outb_v.at[buf], out_hbm.at[h, pl.ds(0, 8), wid], osem
        ).wait()

    fetch_idx(0, 0)
    start_gather(0)

    def body(k, carry):
        h0 = 2 * k
        fetch_idx(h0 + 1, 1)
        start_gather(1)
        wait_gather(0)
        slab_compute(h0, 0)

        @pl.when(k < HIST // 2 - 1)
        def _():
            fetch_idx(h0 + 2, 0)
            start_gather(0)

        wait_gather(1)
        slab_compute(h0 + 1, 1)
        return carry

    lax.fori_loop(0, HIST // 2, body, 0, unroll=False)


def kernel(input, table):
    idx_t_flat = jnp.transpose(input).reshape(-1)
    out5 = _emb_lookup(idx_t_flat, table)
    return jnp.transpose(out5, (2, 4, 0, 1, 3)).reshape(BATCH, HIST, EMBED_DIM)


# speedup vs baseline: 1.6157x; 1.6157x over previous
import functools
import math

import jax
import jax.numpy as jnp
from jax import lax
from jax.experimental import pallas as pl
from jax.experimental.pallas import tpu as pltpu
from jax.experimental.pallas import tpu_sc as plsc

EMBED_DIM = 64
BATCH = 4096
HIST = 200
VOCAB = 1000000
NUM_CORES = 2
NUM_SUBCORES = 16
NW = NUM_CORES * NUM_SUBCORES
BW = BATCH // NW
SCALE = math.sqrt(EMBED_DIM)
L = 16
TBLK = 1024
NTBLK = -(-VOCAB // TBLK)

_mesh = plsc.VectorSubcoreMesh(core_axis_name="c", subcore_axis_name="s")


@functools.partial(
    pl.pallas_call,
    grid=(NTBLK,),
    in_specs=[pl.BlockSpec((EMBED_DIM, TBLK), lambda j: (0, j))],
    out_specs=pl.BlockSpec((TBLK, 128), lambda j: (j, 0)),
    out_shape=jax.ShapeDtypeStruct((VOCAB, 128), jnp.float32),
)
def _repack(tt_ref, out_ref):
    t = jnp.transpose(tt_ref[...])
    out_ref[...] = jnp.concatenate([t, t], axis=1)


@functools.partial(
    pl.kernel,
    mesh=_mesh,
    compiler_params=pltpu.CompilerParams(
        use_tc_tiling_on_sc=False, needs_layout_passes=False),
    out_type=jax.ShapeDtypeStruct((HIST, 8, NW, 8, 128), jnp.float32),
    scratch_types=[
        pltpu.VMEM((HIST // 8, 8, BW), jnp.int32),
        pltpu.VMEM((2, BW, 128), jnp.float32),
        pltpu.VMEM((2, 8, 8, BW), jnp.float32),
        pltpu.SemaphoreType.DMA,
        pltpu.SemaphoreType.DMA,
        pltpu.SemaphoreType.DMA,
        pltpu.SemaphoreType.DMA,
    ],
)
def _emb_lookup(idx_hbm, table_hbm, out_hbm, idx_v, rows_v, outb_v,
                isem, gsem, osem0, osem1):
    wid = lax.axis_index("s") * NUM_CORES + lax.axis_index("c")
    osems = (osem0, osem1)

    pltpu.async_copy(idx_hbm.at[:, wid], idx_v, isem).wait()

    def gather_pair(h, buf):
        th = lax.div(h, 8)
        sh = lax.rem(h, 8)
        return pltpu.async_copy(
            table_hbm.at[idx_v.at[th, sh]], rows_v.at[buf], gsem)

    def start_gather(h, buf):
        gather_pair(h, buf)

    def wait_gather(h, buf):
        pltpu.make_async_copy(
            table_hbm.at[idx_v.at[lax.div(h, 8), lax.rem(h, 8)]],
            rows_v.at[buf], gsem,
        ).wait()

    def slab_compute(h, buf):
        rows = rows_v.at[buf]
        lanes = lax.iota(jnp.int32, L)
        zeros = jnp.zeros((L,), jnp.int32)

        @plsc.parallel_loop(0, 8 * EMBED_DIM, 1, unroll=8)
        def _(j):
            g = lax.shift_right_logical(j, 6)
            d = jnp.bitwise_and(j, EMBED_DIM - 1)
            rowi = lanes + lax.shift_left(g, 4)
            vals = plsc.load_gather(rows, [rowi, zeros + d])
            dt = lax.shift_right_logical(d, 3)
            dl = jnp.bitwise_and(d, 7)
            outb_v[buf, dt, dl, pl.ds(lax.shift_left(g, 4), L)] = vals * SCALE

        pltpu.async_copy(
            outb_v.at[buf], out_hbm.at[h, pl.ds(0, 8), wid], osems[buf])

    def wait_out(h, buf):
        pltpu.make_async_copy(
            outb_v.at[buf], out_hbm.at[h, pl.ds(0, 8), wid], osems[buf]
        ).wait()

    start_gather(0, 0)
    start_gather(1, 1)

    def body(k, carry):
        h0 = 2 * k
        for buf in range(2):
            h = h0 + buf
            wait_gather(h, buf)

            @pl.when(k > 0)
            def _():
                wait_out(h, buf)

            slab_compute(h, buf)

            @pl.when(k < HIST // 2 - 1)
            def _():
                start_gather(h + 2, buf)

        return carry

    lax.fori_loop(0, HIST // 2, body, 0, unroll=False)
    wait_out(HIST - 2, 0)
    wait_out(HIST - 1, 1)


def kernel(input, table):
    idx4 = jnp.transpose(
        input.reshape(NW, BW, HIST // 8, 8), (2, 0, 3, 1))
    table_rm = _repack(jnp.transpose(table))
    out5 = _emb_lookup(idx4, table_rm)
    return jnp.transpose(out5, (2, 4, 0, 1, 3)).reshape(BATCH, HIST, EMBED_DIM)
